# Initial kernel scaffold; baseline (speedup 1.0000x reference)
import jax
import jax.numpy as jnp
from jax.experimental import pallas as pl


def kernel(x, edge_index, edge_attr, W1_rel, b1, W1_root, W2_rel, b2, W2_root, W_lin, b_lin):
    raise NotImplementedError("write your pallas kernel here")


# speedup vs baseline: 14.8300x; 14.8300x over previous
import functools

import jax
import jax.numpy as jnp
from jax import lax
from jax.experimental import pallas as pl
from jax.experimental.pallas import tpu as pltpu
from jax.experimental.pallas import tpu_sc as plsc

N_NODES = 10000
N_EDGES = 640000

NC = 2
NS = 16
NW = NC * NS

NPAD = 10240
IDXC = 128
KPW = 160
KC = 16
EPW = KPW * IDXC
EPAD = NW * EPW
RPW = NPAD // NS


def _make_segsum(d: int, tc_tiling: bool = True):
  mesh = plsc.VectorSubcoreMesh(core_axis_name="c", subcore_axis_name="s")

  @functools.partial(
      pl.kernel,
      mesh=mesh,
      compiler_params=pltpu.CompilerParams(use_tc_tiling_on_sc=tc_tiling),
      out_type=jax.ShapeDtypeStruct((NC * NPAD, d), jnp.float32),
      scratch_types=[
          pltpu.VMEM((KC, IDXC), jnp.int32),
          pltpu.VMEM((KC, IDXC), jnp.int32),
          pltpu.VMEM((IDXC, d), jnp.float32),
          pltpu.VMEM_SHARED((NPAD, d), jnp.float32),
          pltpu.SemaphoreType.DMA,
      ],
  )
  def segsum(table, src3, dst3, zer, out, src_v, dst_v, rows_v, acc, sem):
    c = lax.axis_index("c")
    s = lax.axis_index("s")
    wid = s * NC + c
    pltpu.sync_copy(zer.at[pl.ds(s * RPW, RPW)], acc.at[pl.ds(s * RPW, RPW)])
    plsc.subcore_barrier()

    def outer(j2, carry):
      pltpu.sync_copy(src3.at[wid, pl.ds(j2 * KC, KC)], src_v)
      pltpu.sync_copy(dst3.at[wid, pl.ds(j2 * KC, KC)], dst_v)

      def step(j, carry2):
        pltpu.async_copy(table.at[src_v.at[j]], rows_v, sem).wait()
        pltpu.sync_copy(rows_v, acc.at[dst_v.at[j]], add=True)
        return carry2

      lax.fori_loop(0, KC, step, 0)
      return carry

    lax.fori_loop(0, KPW // KC, outer, 0)
    plsc.subcore_barrier()
    pltpu.sync_copy(acc.at[pl.ds(s * RPW, RPW)],
                    out.at[pl.ds(c * NPAD + s * RPW, RPW)])

  return segsum


_segsum16 = _make_segsum(16, tc_tiling=False)
_segsum128 = _make_segsum(128)


def _layer1_body(aggp, x, w1r, w1o, b1, h1):
  agg = aggp[:NPAD, :] + aggp[NPAD:, :]
  h = (jnp.dot(agg, w1r[...], preferred_element_type=jnp.float32)
       + jnp.dot(x[...], w1o[...], preferred_element_type=jnp.float32)
       + b1[...])
  h1[...] = jnp.maximum(h, 0.0)


def _layer2_body(aggp, h1, w2r, w2o, b2, wl, bl, out):
  agg = aggp[:NPAD, :] + aggp[NPAD:, :]
  z = (jnp.dot(agg, w2r[...], preferred_element_type=jnp.float32)
       + jnp.dot(h1[...], w2o[...], preferred_element_type=jnp.float32)
       + b2[...])
  h2 = jnp.maximum(z, 0.0)
  row = lax.broadcasted_iota(jnp.int32, (NPAD, 1), 0)
  h2 = jnp.where(row < N_NODES, h2, 0.0)
  g = jnp.sum(h2, axis=0, keepdims=True) * (1.0 / N_NODES)
  out[...] = jnp.dot(g, wl[...], preferred_element_type=jnp.float32) + bl[...]


def kernel(x, edge_index, edge_attr, W1_rel, b1, W1_root, W2_rel, b2, W2_root,
           W_lin, b_lin):
  del edge_attr
  f32 = jnp.float32
  x_pad = jnp.zeros((NPAD, 4), f32).at[:N_NODES].set(x)
  npadding = EPAD - N_EDGES
  pad_src = (jnp.arange(npadding, dtype=jnp.int32) * 37) % N_NODES
  pad_dst = N_NODES + (jnp.arange(npadding, dtype=jnp.int32) % (NPAD - N_NODES))
  src3 = jnp.concatenate([edge_index[0], pad_src]).reshape(NW, KPW, IDXC)
  dst3 = jnp.concatenate([edge_index[1], pad_dst]).reshape(NW, KPW, IDXC)
  zer16 = jnp.zeros((NPAD, 16), f32)
  zer128 = jnp.zeros((NPAD, 128), f32)

  x16 = jnp.zeros((NPAD, 16), f32).at[:N_NODES, :4].set(x)
  aggp1 = _segsum16(x16, src3, dst3, zer16)
  w1r16 = jnp.zeros((16, 128), f32).at[:4].set(W1_rel)
  h1 = pl.pallas_call(
      _layer1_body,
      out_shape=jax.ShapeDtypeStruct((NPAD, 128), f32),
  )(aggp1, x_pad, w1r16, W1_root, b1.reshape(1, 128))

  aggp2 = _segsum128(h1, src3, dst3, zer128)
  out = pl.pallas_call(
      _layer2_body,
      out_shape=jax.ShapeDtypeStruct((1, 2), f32),
  )(aggp2, h1, W2_rel, W2_root, b2.reshape(1, 128), W_lin,
    b_lin.reshape(1, 2))
  return out


# speedup vs baseline: 18.1800x; 1.2259x over previous
import functools

import jax
import jax.numpy as jnp
from jax import lax
from jax.experimental import pallas as pl
from jax.experimental.pallas import tpu as pltpu
from jax.experimental.pallas import tpu_sc as plsc

N_NODES = 10000
N_EDGES = 640000

NC = 2
NS = 16
NW = NC * NS

NPAD = 10240
IDXC = 128
KPW = 160
KC = 16
EPW = KPW * IDXC
EPAD = NW * EPW
RPW = NPAD // NS


def _make_segsum(d: int, tc_tiling: bool = True):
  mesh = plsc.VectorSubcoreMesh(core_axis_name="c", subcore_axis_name="s")

  @functools.partial(
      pl.kernel,
      mesh=mesh,
      compiler_params=pltpu.CompilerParams(use_tc_tiling_on_sc=tc_tiling),
      out_type=jax.ShapeDtypeStruct((NC * NPAD, d), jnp.float32),
      scratch_types=[
          pltpu.VMEM((KC, IDXC), jnp.int32),
          pltpu.VMEM((KC, IDXC), jnp.int32),
          pltpu.VMEM((2, IDXC, d), jnp.float32),
          pltpu.VMEM_SHARED((NPAD, d), jnp.float32),
          pltpu.SemaphoreType.DMA,
          pltpu.SemaphoreType.DMA,
      ],
  )
  def segsum(table, src3, dst3, zer, out, src_v, dst_v, rows_v, acc, gsem,
             ssem):
    c = lax.axis_index("c")
    s = lax.axis_index("s")
    wid = s * NC + c
    pltpu.sync_copy(zer.at[pl.ds(s * RPW, RPW)], acc.at[pl.ds(s * RPW, RPW)])
    plsc.subcore_barrier()

    def outer(j2, carry):
      pltpu.sync_copy(src3.at[wid, pl.ds(j2 * KC, KC)], src_v)
      pltpu.sync_copy(dst3.at[wid, pl.ds(j2 * KC, KC)], dst_v)
      gd = [None, None]
      sd = [None] * KC
      gd[0] = pltpu.async_copy(table.at[src_v.at[0]], rows_v.at[0], gsem)
      for j in range(KC):
        b = j % 2
        gd[b].wait()
        if j >= 1:
          sd[j - 1].wait()
        if j + 1 < KC:
          gd[1 - b] = pltpu.async_copy(table.at[src_v.at[j + 1]],
                                       rows_v.at[1 - b], gsem)
        sd[j] = pltpu.async_copy(rows_v.at[b], acc.at[dst_v.at[j]], ssem,
                                 add=True)
      sd[KC - 1].wait()
      return carry

    lax.fori_loop(0, KPW // KC, outer, 0)
    plsc.subcore_barrier()
    pltpu.sync_copy(acc.at[pl.ds(s * RPW, RPW)],
                    out.at[pl.ds(c * NPAD + s * RPW, RPW)])

  return segsum


_segsum16 = _make_segsum(16, tc_tiling=False)
_segsum128 = _make_segsum(128)


def _layer1_body(aggp, x, w1r, w1o, b1, h1):
  agg = aggp[:NPAD, :] + aggp[NPAD:, :]
  h = (jnp.dot(agg, w1r[...], preferred_element_type=jnp.float32)
       + jnp.dot(x[...], w1o[...], preferred_element_type=jnp.float32)
       + b1[...])
  h1[...] = jnp.maximum(h, 0.0)


def _layer2_body(aggp, h1, w2r, w2o, b2, wl, bl, out):
  agg = aggp[:NPAD, :] + aggp[NPAD:, :]
  z = (jnp.dot(agg, w2r[...], preferred_element_type=jnp.float32)
       + jnp.dot(h1[...], w2o[...], preferred_element_type=jnp.float32)
       + b2[...])
  h2 = jnp.maximum(z, 0.0)
  row = lax.broadcasted_iota(jnp.int32, (NPAD, 1), 0)
  h2 = jnp.where(row < N_NODES, h2, 0.0)
  g = jnp.sum(h2, axis=0, keepdims=True) * (1.0 / N_NODES)
  out[...] = jnp.dot(g, wl[...], preferred_element_type=jnp.float32) + bl[...]


def kernel(x, edge_index, edge_attr, W1_rel, b1, W1_root, W2_rel, b2, W2_root,
           W_lin, b_lin):
  del edge_attr
  f32 = jnp.float32
  x_pad = jnp.zeros((NPAD, 4), f32).at[:N_NODES].set(x)
  npadding = EPAD - N_EDGES
  pad_src = (jnp.arange(npadding, dtype=jnp.int32) * 37) % N_NODES
  pad_dst = N_NODES + (jnp.arange(npadding, dtype=jnp.int32) % (NPAD - N_NODES))
  src3 = jnp.concatenate([edge_index[0], pad_src]).reshape(NW, KPW, IDXC)
  dst3 = jnp.concatenate([edge_index[1], pad_dst]).reshape(NW, KPW, IDXC)
  zer16 = jnp.zeros((NPAD, 16), f32)
  zer128 = jnp.zeros((NPAD, 128), f32)

  x16 = jnp.zeros((NPAD, 16), f32).at[:N_NODES, :4].set(x)
  aggp1 = _segsum16(x16, src3, dst3, zer16)
  w1r16 = jnp.zeros((16, 128), f32).at[:4].set(W1_rel)
  h1 = pl.pallas_call(
      _layer1_body,
      out_shape=jax.ShapeDtypeStruct((NPAD, 128), f32),
  )(aggp1, x_pad, w1r16, W1_root, b1.reshape(1, 128))

  aggp2 = _segsum128(h1, src3, dst3, zer128)
  out = pl.pallas_call(
      _layer2_body,
      out_shape=jax.ShapeDtypeStruct((1, 2), f32),
  )(aggp2, h1, W2_rel, W2_root, b2.reshape(1, 128), W_lin,
    b_lin.reshape(1, 2))
  return out


# speedup vs baseline: 30.5377x; 1.6797x over previous
import functools

import jax
import jax.numpy as jnp
from jax import lax
from jax.experimental import pallas as pl
from jax.experimental.pallas import tpu as pltpu
from jax.experimental.pallas import tpu_sc as plsc

N_NODES = 10000
N_EDGES = 640000

NC = 2
NS = 16
NW = NC * NS

NPAD = 10240
IDXC = 128
KPW = 160
KC = 40
EPW = KPW * IDXC
EPAD = NW * EPW
RPW = NPAD // NS


def _make_segsum(d: int, dtype, depth: int, tc_tiling: bool = True):
  mesh = plsc.VectorSubcoreMesh(core_axis_name="c", subcore_axis_name="s")
  nb = 2 * depth

  @functools.partial(
      pl.kernel,
      mesh=mesh,
      compiler_params=pltpu.CompilerParams(use_tc_tiling_on_sc=tc_tiling),
      out_type=jax.ShapeDtypeStruct((NC * NPAD, d), dtype),
      scratch_types=[
          pltpu.VMEM((KC, IDXC), jnp.int32),
          pltpu.VMEM((KC, IDXC), jnp.int32),
          pltpu.VMEM((nb, IDXC, d), dtype),
          pltpu.VMEM_SHARED((NPAD, d), dtype),
          pltpu.SemaphoreType.DMA,
          pltpu.SemaphoreType.DMA,
      ],
  )
  def segsum(table, src3, dst3, zer, out, src_v, dst_v, rows_v, acc, gsem,
             ssem):
    c = lax.axis_index("c")
    s = lax.axis_index("s")
    wid = s * NC + c
    pltpu.sync_copy(zer.at[pl.ds(s * RPW, RPW)], acc.at[pl.ds(s * RPW, RPW)])
    plsc.subcore_barrier()

    def outer(j2, carry):
      pltpu.sync_copy(src3.at[wid, pl.ds(j2 * KC, KC)], src_v)
      pltpu.sync_copy(dst3.at[wid, pl.ds(j2 * KC, KC)], dst_v)
      gd = [None] * KC
      sd = [None] * KC
      for j in range(depth):
        gd[j] = pltpu.async_copy(table.at[src_v.at[j]], rows_v.at[j % nb],
                                 gsem)
      for j in range(KC):
        gd[j].wait()
        if j >= depth:
          sd[j - depth].wait()
        if j + depth < KC:
          gd[j + depth] = pltpu.async_copy(
              table.at[src_v.at[j + depth]], rows_v.at[(j + depth) % nb],
              gsem)
        sd[j] = pltpu.async_copy(rows_v.at[j % nb], acc.at[dst_v.at[j]],
                                 ssem, add=True)
      for j in range(KC - depth, KC):
        sd[j].wait()
      return carry

    lax.fori_loop(0, KPW // KC, outer, 0)
    plsc.subcore_barrier()
    pltpu.sync_copy(acc.at[pl.ds(s * RPW, RPW)],
                    out.at[pl.ds(c * NPAD + s * RPW, RPW)])

  return segsum


_segsum16 = _make_segsum(16, jnp.float32, depth=4, tc_tiling=False)
_segsum128 = _make_segsum(128, jnp.bfloat16, depth=2, tc_tiling=False)


def _layer1_body(aggp, x, w1r, w1o, b1, h1, h1b):
  agg = aggp[:NPAD, :] + aggp[NPAD:, :]
  h = (jnp.dot(agg, w1r[...], preferred_element_type=jnp.float32)
       + jnp.dot(x[...], w1o[...], preferred_element_type=jnp.float32)
       + b1[...])
  h = jnp.maximum(h, 0.0)
  h1[...] = h
  h1b[...] = h.astype(jnp.bfloat16)


def _layer2_body(aggp, h1, w2r, w2o, b2, wl, bl, out):
  agg = (aggp[:NPAD, :].astype(jnp.float32)
         + aggp[NPAD:, :].astype(jnp.float32))
  z = (jnp.dot(agg, w2r[...], preferred_element_type=jnp.float32)
       + jnp.dot(h1[...], w2o[...], preferred_element_type=jnp.float32)
       + b2[...])
  h2 = jnp.maximum(z, 0.0)
  row = lax.broadcasted_iota(jnp.int32, (NPAD, 1), 0)
  h2 = jnp.where(row < N_NODES, h2, 0.0)
  g = jnp.sum(h2, axis=0, keepdims=True) * (1.0 / N_NODES)
  out[...] = jnp.dot(g, wl[...], preferred_element_type=jnp.float32) + bl[...]


def kernel(x, edge_index, edge_attr, W1_rel, b1, W1_root, W2_rel, b2, W2_root,
           W_lin, b_lin):
  del edge_attr
  f32 = jnp.float32
  x_pad = jnp.zeros((NPAD, 4), f32).at[:N_NODES].set(x)
  npadding = EPAD - N_EDGES
  pad_src = (jnp.arange(npadding, dtype=jnp.int32) * 37) % N_NODES
  pad_dst = N_NODES + (jnp.arange(npadding, dtype=jnp.int32) % (NPAD - N_NODES))
  src3 = jnp.concatenate([edge_index[0], pad_src]).reshape(NW, KPW, IDXC)
  dst3 = jnp.concatenate([edge_index[1], pad_dst]).reshape(NW, KPW, IDXC)
  zer16 = jnp.zeros((NPAD, 16), f32)
  zer128 = jnp.zeros((NPAD, 128), jnp.bfloat16)

  x16 = jnp.zeros((NPAD, 16), f32).at[:N_NODES, :4].set(x)
  aggp1 = _segsum16(x16, src3, dst3, zer16)
  w1r16 = jnp.zeros((16, 128), f32).at[:4].set(W1_rel)
  h1, h1b = pl.pallas_call(
      _layer1_body,
      out_shape=(jax.ShapeDtypeStruct((NPAD, 128), f32),
                 jax.ShapeDtypeStruct((NPAD, 128), jnp.bfloat16)),
  )(aggp1, x_pad, w1r16, W1_root, b1.reshape(1, 128))

  aggp2 = _segsum128(h1b, src3, dst3, zer128)
  out = pl.pallas_call(
      _layer2_body,
      out_shape=jax.ShapeDtypeStruct((1, 2), f32),
  )(aggp2, h1, W2_rel, W2_root, b2.reshape(1, 128), W_lin,
    b_lin.reshape(1, 2))
  return out


# speedup vs baseline: 34.9772x; 1.1454x over previous
import functools

import jax
import jax.numpy as jnp
from jax import lax
from jax.experimental import pallas as pl
from jax.experimental.pallas import tpu as pltpu
from jax.experimental.pallas import tpu_sc as plsc

N_NODES = 10000
N_EDGES = 640000

NC = 2
NS = 16
NW = NC * NS

NPAD = 10240
IDXC = 128
KPW = 160
KC = 40
EPW = KPW * IDXC
EPAD = NW * EPW
RPW = NPAD // NS


def _make_segsum(d: int, dtype, depth: int, tc_tiling: bool = True):
  mesh = plsc.VectorSubcoreMesh(core_axis_name="c", subcore_axis_name="s")
  nb = 2 * depth

  @functools.partial(
      pl.kernel,
      mesh=mesh,
      compiler_params=pltpu.CompilerParams(use_tc_tiling_on_sc=tc_tiling),
      out_type=jax.ShapeDtypeStruct((NC * NPAD, d), dtype),
      scratch_types=[
          pltpu.VMEM((KC, IDXC), jnp.int32),
          pltpu.VMEM((KC, IDXC), jnp.int32),
          pltpu.VMEM((nb, IDXC, d), dtype),
          pltpu.VMEM_SHARED((NPAD, d), dtype),
          pltpu.SemaphoreType.DMA,
          pltpu.SemaphoreType.DMA,
      ],
  )
  def segsum(table, src3, dst3, zer, out, src_v, dst_v, rows_v, acc, gsem,
             ssem):
    c = lax.axis_index("c")
    s = lax.axis_index("s")
    wid = s * NC + c
    pltpu.sync_copy(zer, rows_v.at[0])
    for t in range(RPW // IDXC):
      pltpu.sync_copy(rows_v.at[0],
                      acc.at[pl.ds(s * RPW + t * IDXC, IDXC)])
    plsc.subcore_barrier()

    def outer(j2, carry):
      pltpu.sync_copy(src3.at[wid, pl.ds(j2 * KC, KC)], src_v)
      pltpu.sync_copy(dst3.at[wid, pl.ds(j2 * KC, KC)], dst_v)
      gd = [None] * KC
      sd = [None] * KC
      for j in range(depth):
        gd[j] = pltpu.async_copy(table.at[src_v.at[j]], rows_v.at[j % nb],
                                 gsem)
      for j in range(KC):
        gd[j].wait()
        if j >= depth:
          sd[j - depth].wait()
        if j + depth < KC:
          gd[j + depth] = pltpu.async_copy(
              table.at[src_v.at[j + depth]], rows_v.at[(j + depth) % nb],
              gsem)
        sd[j] = pltpu.async_copy(rows_v.at[j % nb], acc.at[dst_v.at[j]],
                                 ssem, add=True)
      for j in range(KC - depth, KC):
        sd[j].wait()
      return carry

    lax.fori_loop(0, KPW // KC, outer, 0)
    plsc.subcore_barrier()
    pltpu.sync_copy(acc.at[pl.ds(s * RPW, RPW)],
                    out.at[pl.ds(c * NPAD + s * RPW, RPW)])

  return segsum


_segsum16 = _make_segsum(16, jnp.float32, depth=8, tc_tiling=False)
_segsum128 = _make_segsum(128, jnp.bfloat16, depth=3, tc_tiling=False)


def _layer1_body(aggp, x, w1r, w1o, b1, h1, h1b):
  agg = aggp[:NPAD, :] + aggp[NPAD:, :]
  h = (jnp.dot(agg, w1r[...], preferred_element_type=jnp.float32)
       + jnp.dot(x[...], w1o[...], preferred_element_type=jnp.float32)
       + b1[...])
  h = jnp.maximum(h, 0.0)
  h1[...] = h
  h1b[...] = h.astype(jnp.bfloat16)


def _layer2_body(aggp, h1, w2r, w2o, b2, wl, bl, out):
  agg = (aggp[:NPAD, :].astype(jnp.float32)
         + aggp[NPAD:, :].astype(jnp.float32))
  z = (jnp.dot(agg, w2r[...], preferred_element_type=jnp.float32)
       + jnp.dot(h1[...], w2o[...], preferred_element_type=jnp.float32)
       + b2[...])
  h2 = jnp.maximum(z, 0.0)
  row = lax.broadcasted_iota(jnp.int32, (NPAD, 1), 0)
  h2 = jnp.where(row < N_NODES, h2, 0.0)
  g = jnp.sum(h2, axis=0, keepdims=True) * (1.0 / N_NODES)
  out[...] = jnp.dot(g, wl[...], preferred_element_type=jnp.float32) + bl[...]


def kernel(x, edge_index, edge_attr, W1_rel, b1, W1_root, W2_rel, b2, W2_root,
           W_lin, b_lin):
  del edge_attr
  f32 = jnp.float32
  x_pad = jnp.zeros((NPAD, 4), f32).at[:N_NODES].set(x)
  npadding = EPAD - N_EDGES
  pad_src = (jnp.arange(npadding, dtype=jnp.int32) * 37) % N_NODES
  pad_dst = N_NODES + (jnp.arange(npadding, dtype=jnp.int32) % (NPAD - N_NODES))
  src3 = jnp.concatenate([edge_index[0], pad_src]).reshape(NW, KPW, IDXC)
  dst3 = jnp.concatenate([edge_index[1], pad_dst]).reshape(NW, KPW, IDXC)
  zer16 = jnp.zeros((IDXC, 16), f32)
  zer128 = jnp.zeros((IDXC, 128), jnp.bfloat16)

  x16 = jnp.zeros((NPAD, 16), f32).at[:N_NODES, :4].set(x)
  aggp1 = _segsum16(x16, src3, dst3, zer16)
  w1r16 = jnp.zeros((16, 128), f32).at[:4].set(W1_rel)
  h1, h1b = pl.pallas_call(
      _layer1_body,
      out_shape=(jax.ShapeDtypeStruct((NPAD, 128), f32),
                 jax.ShapeDtypeStruct((NPAD, 128), jnp.bfloat16)),
  )(aggp1, x_pad, w1r16, W1_root, b1.reshape(1, 128))

  aggp2 = _segsum128(h1b, src3, dst3, zer128)
  out = pl.pallas_call(
      _layer2_body,
      out_shape=jax.ShapeDtypeStruct((1, 2), f32),
  )(aggp2, h1, W2_rel, W2_root, b2.reshape(1, 128), W_lin,
    b_lin.reshape(1, 2))
  return out
